# baseline (speedup 1.0000x reference)
import functools

import jax
import jax.numpy as jnp
from jax import lax
from jax.experimental import pallas as pl
from jax.experimental.pallas import tpu as pltpu
from jax.experimental.pallas import tpu_sc as plsc

D_MODEL = 64
_LANES = 16

_info = plsc.get_sparse_core_info()
_NC = _info.num_cores
_NS = _info.num_subcores
_NW = _NC * _NS

_SR = 2
_RPS = _SR * 128


def _gather_kernel(table_hbm, tok_hbm, out_hbm,
                   tok_v, idx0_v, idx1_v, hid_v, rows_v, ob_v,
                   sem_t, sem_g0, sem_g1, sem_o0, sem_o1, *, n_steps):
    wid = lax.axis_index("s") * _NC + lax.axis_index("c")
    tc = wid
    sems_g = [sem_g0, sem_g1]
    sems_o = [sem_o0, sem_o1]

    lane = lax.iota(jnp.int32, _LANES)
    row_base = [lane + jb * _LANES for jb in range(2 * 8)]

    def load_tok_tile(tr):
        cp = pltpu.make_async_copy(
            tok_hbm.at[pl.ds(tr * 8, 8), pl.ds(tc * 128, 128)], tok_v, sem_t)
        cp.start()
        cp.wait()

    def prep_step(step):
        buf = step % 2

        @pl.when(step % 4 == 0)
        def _():
            load_tok_tile(step // 4)

        r0 = (step % 4) * _SR
        bufv = jnp.zeros((_LANES,), jnp.int32) + buf
        for sr in range(_SR):
            row = jnp.zeros((_LANES,), jnp.int32) + (r0 + sr)
            for k in range(8):
                t16 = plsc.load_gather(tok_v, [row, lane + k * _LANES])
                col = lane + (sr * 8 + k) * _LANES
                pair = lax.shift_right_logical(t16, 1)

                @pl.when(buf == 0)
                def _():
                    plsc.store_scatter(idx0_v, [col], pair)

                @pl.when(buf == 1)
                def _():
                    plsc.store_scatter(idx1_v, [col], pair)

                plsc.store_scatter(hid_v, [bufv * _RPS + col], lax.shift_left(
                    lax.bitwise_and(t16, jnp.int32(1)), 6))

    def issue_gather(step):
        buf = step % 2

        @pl.when(buf == 0)
        def _():
            pltpu.async_copy(table_hbm.at[idx0_v],
                             rows_v.at[pl.ds(0, _RPS)], sems_g[0])

        @pl.when(buf == 1)
        def _():
            pltpu.async_copy(table_hbm.at[idx1_v],
                             rows_v.at[pl.ds(_RPS, _RPS)], sems_g[1])

    def wait_gather(step):
        buf = step % 2

        @pl.when(buf == 0)
        def _():
            pltpu.make_async_copy(table_hbm.at[idx0_v],
                                  rows_v.at[pl.ds(0, _RPS)], sems_g[0]).wait()

        @pl.when(buf == 1)
        def _():
            pltpu.make_async_copy(table_hbm.at[idx1_v],
                                  rows_v.at[pl.ds(_RPS, _RPS)],
                                  sems_g[1]).wait()

    def out_dma_p(step, parity):
        return pltpu.make_async_copy(
            ob_v.at[pl.ds(parity * _SR, _SR)],
            out_hbm.at[pl.ds(step * _SR, _SR), pl.ds(0, D_MODEL),
                       pl.ds(tc * 128, 128)],
            sems_o[parity])

    def out_start(step):
        buf = step % 2

        @pl.when(buf == 0)
        def _():
            out_dma_p(step, 0).start()

        @pl.when(buf == 1)
        def _():
            out_dma_p(step, 1).start()

    def out_wait(step):
        buf = step % 2

        @pl.when(buf == 0)
        def _():
            out_dma_p(step, 0).wait()

        @pl.when(buf == 1)
        def _():
            out_dma_p(step, 1).wait()

    def assemble(step):
        buf = step % 2
        bufv = jnp.zeros((_LANES,), jnp.int32) + buf
        rb = [row_base[j] + buf * _RPS for j in range(16)]
        hb = [plsc.load_gather(hid_v, [bufv * _RPS + (lane + j * _LANES)])
              for j in range(16)]

        @plsc.parallel_loop(0, 8, unroll=4)
        def loop_d(d8):
            for dk in range(8):
                d = d8 * 8 + dk
                dv = bufv * 0 + d
                for sr in range(_SR):
                    bsv = bufv * _SR + sr
                    for jb in range(8):
                        j = sr * 8 + jb
                        vals = plsc.load_gather(
                            rows_v, [rb[j], hb[j] + d])
                        plsc.store_scatter(
                            ob_v, [bsv, dv, lane + jb * _LANES], vals)

    prep_step(0)
    issue_gather(0)

    def loop_step(step, carry):
        @pl.when(step + 1 < n_steps)
        def _():
            prep_step(step + 1)
            issue_gather(step + 1)

        wait_gather(step)

        @pl.when(step >= 2)
        def _():
            out_wait(step - 2)

        assemble(step)
        out_start(step)
        return carry

    lax.fori_loop(0, n_steps, loop_step, 0, unroll=False)

    out_dma_p(n_steps - 2, (n_steps - 2) % 2).wait()
    out_dma_p(n_steps - 1, (n_steps - 1) % 2).wait()


def kernel(tokens, embeddings):
    b_tok, s_tok = tokens.shape
    tok_t = tokens.T.astype(jnp.int32)
    table2 = embeddings.reshape(-1, 128)
    n_steps = s_tok // _SR

    mesh = plsc.VectorSubcoreMesh(core_axis_name="c", subcore_axis_name="s")
    run = pl.kernel(
        functools.partial(_gather_kernel, n_steps=n_steps),
        mesh=mesh,
        out_type=jax.ShapeDtypeStruct((s_tok, D_MODEL, b_tok), jnp.float32),
        scratch_types=[
            pltpu.VMEM((8, 128), jnp.int32),
            pltpu.VMEM((_RPS,), jnp.int32),
            pltpu.VMEM((_RPS,), jnp.int32),
            pltpu.VMEM((2 * _RPS,), jnp.int32),
            pltpu.VMEM((2 * _RPS, 128), jnp.float32),
            pltpu.VMEM((2 * _SR, D_MODEL, 128), jnp.float32),
            pltpu.SemaphoreType.DMA,
            pltpu.SemaphoreType.DMA,
            pltpu.SemaphoreType.DMA,
            pltpu.SemaphoreType.DMA,
            pltpu.SemaphoreType.DMA,
        ],
        compiler_params=pltpu.CompilerParams(
            use_tc_tiling_on_sc=True, needs_layout_passes=False),
    )
    out_t = run(table2, tok_t)
    return out_t.transpose(2, 0, 1)


# speedup vs baseline: 1.0739x; 1.0739x over previous
import functools

import jax
import jax.numpy as jnp
from jax import lax
from jax.experimental import pallas as pl
from jax.experimental.pallas import tpu as pltpu
from jax.experimental.pallas import tpu_sc as plsc

D_MODEL = 64
_LANES = 16

_info = plsc.get_sparse_core_info()
_NC = _info.num_cores
_NS = _info.num_subcores
_NW = _NC * _NS

_SR = 2
_RPS = _SR * 128


def _gather_kernel(table_hbm, tok_hbm, out_hbm,
                   tok_v, idx0_v, idx1_v, hid_v, rows_v, ob_v,
                   sem_t, sem_g0, sem_g1, sem_o0, sem_o1, *, n_steps):
    wid = lax.axis_index("s") * _NC + lax.axis_index("c")
    tc = wid
    sems_g = [sem_g0, sem_g1]
    sems_o = [sem_o0, sem_o1]

    lane = lax.iota(jnp.int32, _LANES)
    row_base = [lane + jb * _LANES for jb in range(2 * 8)]

    def load_tok_tile(tr):
        cp = pltpu.make_async_copy(
            tok_hbm.at[pl.ds(tr * 8, 8), pl.ds(tc * 128, 128)], tok_v, sem_t)
        cp.start()
        cp.wait()

    def prep_step(step):
        buf = step % 2

        @pl.when(step % 4 == 0)
        def _():
            load_tok_tile(step // 4)

        r0 = (step % 4) * _SR
        bufv = jnp.zeros((_LANES,), jnp.int32) + buf
        for sr in range(_SR):
            row = jnp.zeros((_LANES,), jnp.int32) + (r0 + sr)
            for k in range(8):
                t16 = plsc.load_gather(tok_v, [row, lane + k * _LANES])
                col = lane + (sr * 8 + k) * _LANES
                pair = lax.shift_right_logical(t16, 1)

                @pl.when(buf == 0)
                def _():
                    plsc.store_scatter(idx0_v, [col], pair)

                @pl.when(buf == 1)
                def _():
                    plsc.store_scatter(idx1_v, [col], pair)

                plsc.store_scatter(hid_v, [bufv * _RPS + col], lax.shift_left(
                    lax.bitwise_and(t16, jnp.int32(1)), 6))

    def issue_gather(step):
        buf = step % 2

        @pl.when(buf == 0)
        def _():
            pltpu.async_copy(table_hbm.at[idx0_v],
                             rows_v.at[pl.ds(0, _RPS)], sems_g[0])

        @pl.when(buf == 1)
        def _():
            pltpu.async_copy(table_hbm.at[idx1_v],
                             rows_v.at[pl.ds(_RPS, _RPS)], sems_g[1])

    def wait_gather(step):
        buf = step % 2

        @pl.when(buf == 0)
        def _():
            pltpu.make_async_copy(table_hbm.at[idx0_v],
                                  rows_v.at[pl.ds(0, _RPS)], sems_g[0]).wait()

        @pl.when(buf == 1)
        def _():
            pltpu.make_async_copy(table_hbm.at[idx1_v],
                                  rows_v.at[pl.ds(_RPS, _RPS)],
                                  sems_g[1]).wait()

    def out_dma_p(step, parity):
        return pltpu.make_async_copy(
            ob_v.at[pl.ds(parity * _SR, _SR)],
            out_hbm.at[pl.ds(step * _SR, _SR), pl.ds(0, D_MODEL),
                       pl.ds(tc * 128, 128)],
            sems_o[parity])

    def out_start(step):
        buf = step % 2

        @pl.when(buf == 0)
        def _():
            out_dma_p(step, 0).start()

        @pl.when(buf == 1)
        def _():
            out_dma_p(step, 1).start()

    def out_wait(step):
        buf = step % 2

        @pl.when(buf == 0)
        def _():
            out_dma_p(step, 0).wait()

        @pl.when(buf == 1)
        def _():
            out_dma_p(step, 1).wait()

    def assemble(step):
        buf = step % 2
        bufv = jnp.zeros((_LANES,), jnp.int32) + buf
        rb = [row_base[j] + buf * _RPS for j in range(16)]
        hb = [plsc.load_gather(hid_v, [bufv * _RPS + (lane + j * _LANES)])
              for j in range(16)]

        @plsc.parallel_loop(0, 8, unroll=2)
        def loop_d(d8):
            for dk in range(8):
                d = d8 * 8 + dk
                dv = bufv * 0 + d
                for sr in range(_SR):
                    bsv = bufv * _SR + sr
                    for jb in range(8):
                        j = sr * 8 + jb
                        vals = plsc.load_gather(
                            rows_v, [rb[j], hb[j] + d])
                        plsc.store_scatter(
                            ob_v, [bsv, dv, lane + jb * _LANES], vals)

    prep_step(0)
    issue_gather(0)

    def loop_step(step, carry):
        @pl.when(step + 1 < n_steps)
        def _():
            prep_step(step + 1)
            issue_gather(step + 1)

        wait_gather(step)

        @pl.when(step >= 2)
        def _():
            out_wait(step - 2)

        assemble(step)
        out_start(step)
        return carry

    lax.fori_loop(0, n_steps, loop_step, 0, unroll=False)

    out_dma_p(n_steps - 2, (n_steps - 2) % 2).wait()
    out_dma_p(n_steps - 1, (n_steps - 1) % 2).wait()


def kernel(tokens, embeddings):
    b_tok, s_tok = tokens.shape
    tok_t = tokens.T.astype(jnp.int32)
    table2 = embeddings.reshape(-1, 128)
    n_steps = s_tok // _SR

    mesh = plsc.VectorSubcoreMesh(core_axis_name="c", subcore_axis_name="s")
    run = pl.kernel(
        functools.partial(_gather_kernel, n_steps=n_steps),
        mesh=mesh,
        out_type=jax.ShapeDtypeStruct((s_tok, D_MODEL, b_tok), jnp.float32),
        scratch_types=[
            pltpu.VMEM((8, 128), jnp.int32),
            pltpu.VMEM((_RPS,), jnp.int32),
            pltpu.VMEM((_RPS,), jnp.int32),
            pltpu.VMEM((2 * _RPS,), jnp.int32),
            pltpu.VMEM((2 * _RPS, 128), jnp.float32),
            pltpu.VMEM((2 * _SR, D_MODEL, 128), jnp.float32),
            pltpu.SemaphoreType.DMA,
            pltpu.SemaphoreType.DMA,
            pltpu.SemaphoreType.DMA,
            pltpu.SemaphoreType.DMA,
            pltpu.SemaphoreType.DMA,
        ],
        compiler_params=pltpu.CompilerParams(
            use_tc_tiling_on_sc=True, needs_layout_passes=False),
    )
    out_t = run(table2, tok_t)
    return out_t.transpose(2, 0, 1)


# speedup vs baseline: 1.1973x; 1.1149x over previous
import functools

import jax
import jax.numpy as jnp
from jax import lax
from jax.experimental import pallas as pl
from jax.experimental.pallas import tpu as pltpu
from jax.experimental.pallas import tpu_sc as plsc

D_MODEL = 64

_info = plsc.get_sparse_core_info()
_NC = _info.num_cores
_NS = _info.num_subcores
_NW = _NC * _NS

_CHUNK = 640
_NBUF = 2


def _gather_kernel(table_hbm, idx_hbm, out_hbm, idx_all, rows,
                   sem_i, sem_g0, sem_g1, sem_o0, sem_o1,
                   *, b_per_w, n_chunks):
    wid = lax.axis_index("s") * _NC + lax.axis_index("c")
    base = wid * b_per_w
    sems_g = [sem_g0, sem_g1]
    sems_o = [sem_o0, sem_o1]

    cp_i = pltpu.make_async_copy(
        idx_hbm.at[pl.ds(base, b_per_w)], idx_all, sem_i)
    cp_i.start()
    cp_i.wait()

    def gather_src(i):
        return table_hbm.at[idx_all.at[pl.ds(i * _CHUNK, _CHUNK)]]

    def gather_cp(i, parity):
        return pltpu.make_async_copy(
            gather_src(i), rows.at[pl.ds(parity * _CHUNK, _CHUNK)],
            sems_g[parity])

    def out_cp(i, parity):
        return pltpu.make_async_copy(
            rows.at[pl.ds(parity * _CHUNK, _CHUNK)],
            out_hbm.at[pl.ds(base + i * _CHUNK, _CHUNK)],
            sems_o[parity])

    def g_start(i):
        @pl.when(i % 2 == 0)
        def _():
            gather_cp(i, 0).start()

        @pl.when(i % 2 == 1)
        def _():
            gather_cp(i, 1).start()

    def g_wait(i):
        @pl.when(i % 2 == 0)
        def _():
            gather_cp(i, 0).wait()

        @pl.when(i % 2 == 1)
        def _():
            gather_cp(i, 1).wait()

    def o_start(i):
        @pl.when(i % 2 == 0)
        def _():
            out_cp(i, 0).start()

        @pl.when(i % 2 == 1)
        def _():
            out_cp(i, 1).start()

    def o_wait(i):
        @pl.when(i % 2 == 0)
        def _():
            out_cp(i, 0).wait()

        @pl.when(i % 2 == 1)
        def _():
            out_cp(i, 1).wait()

    gather_cp(0, 0).start()

    def step(i, carry):
        g_wait(i)

        @pl.when(i >= 2)
        def _():
            o_wait(i - 2)

        @pl.when(i + 1 < n_chunks)
        def _():
            g_start(i + 1)

        o_start(i)
        return carry

    lax.fori_loop(0, n_chunks, step, 0, unroll=False)

    out_cp(n_chunks - 2, (n_chunks - 2) % 2).wait()
    out_cp(n_chunks - 1, (n_chunks - 1) % 2).wait()


def kernel(tokens, embeddings):
    orig_shape = tokens.shape
    idx = tokens.reshape(-1).astype(jnp.int32)
    b = idx.shape[0]
    b_per_w = b // _NW
    n_chunks = b_per_w // _CHUNK

    mesh = plsc.VectorSubcoreMesh(core_axis_name="c", subcore_axis_name="s")
    run = pl.kernel(
        functools.partial(_gather_kernel, b_per_w=b_per_w, n_chunks=n_chunks),
        mesh=mesh,
        out_type=jax.ShapeDtypeStruct((b, D_MODEL), jnp.float32),
        scratch_types=[
            pltpu.VMEM((b_per_w,), jnp.int32),
            pltpu.VMEM((_NBUF * _CHUNK, D_MODEL), jnp.float32),
            pltpu.SemaphoreType.DMA,
            pltpu.SemaphoreType.DMA,
            pltpu.SemaphoreType.DMA,
            pltpu.SemaphoreType.DMA,
            pltpu.SemaphoreType.DMA,
        ],
        compiler_params=pltpu.CompilerParams(
            use_tc_tiling_on_sc=False, needs_layout_passes=False),
    )
    out = run(embeddings, idx)
    return out.reshape(orig_shape + (D_MODEL,))
